# baseline (speedup 1.0000x reference)
import functools

import jax
import jax.numpy as jnp
from jax import lax
from jax.experimental import pallas as pl
from jax.experimental.pallas import tpu as pltpu
from jax.experimental.pallas import tpu_sc as plsc

BATCH = 4096
HIST = 200
DIM = 64
PDIM = 128
NUM_IDX = BATCH * HIST
NUM_CORES = 2
NUM_SUBCORES = 16
NW = NUM_CORES * NUM_SUBCORES
PER_W = NUM_IDX // NW
CHUNK = 128
N_CHUNKS = PER_W // CHUNK
K = 2
N_GROUPS = N_CHUNKS // K


def _emb_body(idx_hbm, table_hbm, out_hbm, idx_v, rows_v, gsem, ssem):
    cid = lax.axis_index("c")
    sid = lax.axis_index("s")
    wid = sid * NUM_CORES + cid
    base = wid * PER_W

    pltpu.sync_copy(idx_hbm.at[wid], idx_v)

    def fire_gathers(jg, half):
        for b in range(K):
            pltpu.async_copy(
                table_hbm.at[idx_v.at[jg + b]], rows_v.at[half, b], gsem)

    def drain_gathers(half):
        for b in range(K):
            pltpu.make_async_copy(
                table_hbm.at[pl.ds(0, CHUNK)], rows_v.at[half, b], gsem).wait()

    def fire_scatters(jg, half):
        for b in range(K):
            pltpu.async_copy(
                rows_v.at[half, b, :, pl.ds(0, DIM)],
                out_hbm.at[pl.ds(base + (jg + b) * CHUNK, CHUNK)], ssem)

    def drain_scatters(half):
        for b in range(K):
            pltpu.make_async_copy(
                rows_v.at[half, b, :, pl.ds(0, DIM)],
                out_hbm.at[pl.ds(base, CHUNK)], ssem).wait()

    fire_gathers(0, 0)

    def body(g, carry):
        half = lax.rem(g, 2)
        other = 1 - half
        jg = g * K
        drain_gathers(half)

        @pl.when(g >= 1)
        def _():
            drain_scatters(other)

        @pl.when(g < N_GROUPS - 1)
        def _():
            fire_gathers(jg + K, other)

        fire_scatters(jg, half)
        return carry

    lax.fori_loop(0, N_GROUPS, body, 0)
    drain_scatters((N_GROUPS - 1) % 2)


@functools.partial(
    pl.kernel,
    mesh=plsc.VectorSubcoreMesh(core_axis_name="c", subcore_axis_name="s"),
    compiler_params=pltpu.CompilerParams(use_tc_tiling_on_sc=False),
    out_type=jax.ShapeDtypeStruct((NUM_IDX, DIM), jnp.float32),
    scratch_types=[
        pltpu.VMEM((N_CHUNKS, CHUNK), jnp.int32),
        pltpu.VMEM((2, K, CHUNK, PDIM), jnp.float32),
        pltpu.SemaphoreType.DMA,
        pltpu.SemaphoreType.DMA,
    ],
)
def _gather(idx_hbm, table_hbm, out_hbm, idx_v, rows_v, gsem, ssem):
    _emb_body(idx_hbm, table_hbm, out_hbm, idx_v, rows_v, gsem, ssem)


def kernel(skill_ids, embeddings):
    idx = skill_ids.reshape(NW, N_CHUNKS, CHUNK).astype(jnp.int32)
    tpad = jnp.pad(embeddings, ((0, 0), (0, PDIM - DIM)))
    out = _gather(idx, tpad)
    return out.reshape(BATCH, HIST, DIM)


# speedup vs baseline: 1.0148x; 1.0148x over previous
import functools

import jax
import jax.numpy as jnp
from jax import lax
from jax.experimental import pallas as pl
from jax.experimental.pallas import tpu as pltpu
from jax.experimental.pallas import tpu_sc as plsc

BATCH = 4096
HIST = 200
DIM = 64
NUM_IDX = BATCH * HIST
NUM_CORES = 2
NUM_SUBCORES = 16
NW = NUM_CORES * NUM_SUBCORES
PER_W = NUM_IDX // NW
CHUNK = 128
N_CHUNKS = PER_W // CHUNK
K = 2
N_GROUPS = N_CHUNKS // K


def _emb_body(idx_hbm, table_hbm, out_hbm, idx_v, rows_v, gsem, ssem):
    cid = lax.axis_index("c")
    sid = lax.axis_index("s")
    wid = sid * NUM_CORES + cid
    base = wid * PER_W

    pltpu.sync_copy(idx_hbm.at[wid], idx_v)

    def fire_gathers(jg, half):
        for b in range(K):
            pltpu.async_copy(
                table_hbm.at[idx_v.at[jg + b]], rows_v.at[half, b], gsem)

    def drain_gathers(half):
        for b in range(K):
            pltpu.make_async_copy(
                table_hbm.at[pl.ds(0, CHUNK)], rows_v.at[half, b], gsem).wait()

    def fire_scatters(jg, half):
        for b in range(K):
            pltpu.async_copy(
                rows_v.at[half, b],
                out_hbm.at[pl.ds(base + (jg + b) * CHUNK, CHUNK)], ssem)

    def drain_scatters(half):
        for b in range(K):
            pltpu.make_async_copy(
                rows_v.at[half, b],
                out_hbm.at[pl.ds(base, CHUNK)], ssem).wait()

    fire_gathers(0, 0)

    def body(g, carry):
        half = lax.rem(g, 2)
        other = 1 - half
        jg = g * K
        drain_gathers(half)

        @pl.when(g >= 1)
        def _():
            drain_scatters(other)

        @pl.when(g < N_GROUPS - 1)
        def _():
            fire_gathers(jg + K, other)

        fire_scatters(jg, half)
        return carry

    lax.fori_loop(0, N_GROUPS, body, 0)
    drain_scatters((N_GROUPS - 1) % 2)


@functools.partial(
    pl.kernel,
    mesh=plsc.VectorSubcoreMesh(core_axis_name="c", subcore_axis_name="s"),
    compiler_params=pltpu.CompilerParams(use_tc_tiling_on_sc=False),
    out_type=jax.ShapeDtypeStruct((NUM_IDX, DIM), jnp.float32),
    scratch_types=[
        pltpu.VMEM((N_CHUNKS, CHUNK), jnp.int32),
        pltpu.VMEM((2, K, CHUNK, DIM), jnp.float32),
        pltpu.SemaphoreType.DMA,
        pltpu.SemaphoreType.DMA,
    ],
)
def _gather(idx_hbm, table_hbm, out_hbm, idx_v, rows_v, gsem, ssem):
    _emb_body(idx_hbm, table_hbm, out_hbm, idx_v, rows_v, gsem, ssem)


def kernel(skill_ids, embeddings):
    idx = skill_ids.reshape(NW, N_CHUNKS, CHUNK).astype(jnp.int32)
    out = _gather(idx, embeddings)
    return out.reshape(BATCH, HIST, DIM)


# speedup vs baseline: 1.2485x; 1.2303x over previous
import functools

import jax
import jax.numpy as jnp
from jax import lax
from jax.experimental import pallas as pl
from jax.experimental.pallas import tpu as pltpu
from jax.experimental.pallas import tpu_sc as plsc

BATCH = 4096
HIST = 200
DIM = 64
PDIM = 128
NUM_IDX = BATCH * HIST
NUM_CORES = 2
NUM_SUBCORES = 16
NW = NUM_CORES * NUM_SUBCORES
PER_W = BATCH // NW
PER_W_IDX = PER_W * HIST


def _emb_body(idx_hbm, table_hbm, out_hbm, idx_v, rows_v, gsem, ssem):
    cid = lax.axis_index("c")
    sid = lax.axis_index("s")
    wid = sid * NUM_CORES + cid
    base = wid * PER_W

    pltpu.sync_copy(idx_hbm.at[wid], idx_v)

    def fire_gather(b, half):
        pltpu.async_copy(
            table_hbm.at[idx_v.at[pl.ds(b * HIST, HIST)]], rows_v.at[half],
            gsem)

    def drain_gather(half):
        pltpu.make_async_copy(
            table_hbm.at[pl.ds(0, HIST)], rows_v.at[half], gsem).wait()

    def fire_scatter(b, half):
        pltpu.async_copy(rows_v.at[half], out_hbm.at[base + b], ssem)

    def drain_scatter(half):
        pltpu.make_async_copy(
            rows_v.at[half], out_hbm.at[base], ssem).wait()

    fire_gather(0, 0)

    def body(b, carry):
        half = lax.rem(b, 2)
        other = 1 - half
        drain_gather(half)

        @pl.when(b >= 1)
        def _():
            drain_scatter(other)

        @pl.when(b < PER_W - 1)
        def _():
            fire_gather(b + 1, other)

        fire_scatter(b, half)
        return carry

    lax.fori_loop(0, PER_W, body, 0)
    drain_scatter((PER_W - 1) % 2)


@functools.partial(
    pl.kernel,
    mesh=plsc.VectorSubcoreMesh(core_axis_name="c", subcore_axis_name="s"),
    compiler_params=pltpu.CompilerParams(use_tc_tiling_on_sc=True),
    out_type=jax.ShapeDtypeStruct((BATCH, HIST, PDIM), jnp.float32),
    scratch_types=[
        pltpu.VMEM((PER_W_IDX,), jnp.int32),
        pltpu.VMEM((2, HIST, PDIM), jnp.float32),
        pltpu.SemaphoreType.DMA,
        pltpu.SemaphoreType.DMA,
    ],
)
def _gather(idx_hbm, table_hbm, out_hbm, idx_v, rows_v, gsem, ssem):
    _emb_body(idx_hbm, table_hbm, out_hbm, idx_v, rows_v, gsem, ssem)


def kernel(skill_ids, embeddings):
    idx = skill_ids.reshape(NW, PER_W_IDX).astype(jnp.int32)
    tpad = jnp.pad(embeddings, ((0, 0), (0, PDIM - DIM)))
    return _gather(idx, tpad)[..., :DIM]


# speedup vs baseline: 1.2653x; 1.0135x over previous
import functools

import jax
import jax.numpy as jnp
from jax import lax
from jax.experimental import pallas as pl
from jax.experimental.pallas import tpu as pltpu
from jax.experimental.pallas import tpu_sc as plsc

MAX_ROWS = 1000000
BATCH = 4096
HIST = 200
DIM = 64
PDIM = 128
NUM_IDX = BATCH * HIST
NUM_CORES = 2
NUM_SUBCORES = 16
NW = NUM_CORES * NUM_SUBCORES
PER_W_IDX = NUM_IDX // NW
ROWS = 400
N_STREAMS = PER_W_IDX // ROWS


def _emb_body(idx_hbm, table_hbm, out_hbm, idx_v, rows_v, gsem, ssem):
    cid = lax.axis_index("c")
    sid = lax.axis_index("s")
    wid = sid * NUM_CORES + cid
    base = wid * PER_W_IDX

    pltpu.sync_copy(idx_hbm.at[wid], idx_v)

    def fire_gather(b, half):
        pltpu.async_copy(
            table_hbm.at[idx_v.at[pl.ds(b * ROWS, ROWS)]], rows_v.at[half],
            gsem)

    def drain_gather(half):
        pltpu.make_async_copy(
            table_hbm.at[pl.ds(0, ROWS)], rows_v.at[half], gsem).wait()

    def fire_scatter(b, half):
        pltpu.async_copy(
            rows_v.at[half], out_hbm.at[pl.ds(base + b * ROWS, ROWS)], ssem)

    def drain_scatter(half):
        pltpu.make_async_copy(
            rows_v.at[half], out_hbm.at[pl.ds(base, ROWS)], ssem).wait()

    fire_gather(0, 0)

    def body(b, carry):
        half = lax.rem(b, 2)
        other = 1 - half
        drain_gather(half)

        @pl.when(b >= 1)
        def _():
            drain_scatter(other)

        @pl.when(b < N_STREAMS - 1)
        def _():
            fire_gather(b + 1, other)

        fire_scatter(b, half)
        return carry

    lax.fori_loop(0, N_STREAMS, body, 0)
    drain_scatter((N_STREAMS - 1) % 2)


@functools.partial(
    pl.kernel,
    mesh=plsc.VectorSubcoreMesh(core_axis_name="c", subcore_axis_name="s"),
    compiler_params=pltpu.CompilerParams(use_tc_tiling_on_sc=True),
    out_type=jax.ShapeDtypeStruct((NUM_IDX, PDIM), jnp.float32),
    scratch_types=[
        pltpu.VMEM((PER_W_IDX,), jnp.int32),
        pltpu.VMEM((2, ROWS, PDIM), jnp.float32),
        pltpu.SemaphoreType.DMA,
        pltpu.SemaphoreType.DMA,
    ],
)
def _gather(idx_hbm, table_hbm, out_hbm, idx_v, rows_v, gsem, ssem):
    _emb_body(idx_hbm, table_hbm, out_hbm, idx_v, rows_v, gsem, ssem)


def kernel(skill_ids, embeddings):
    idx = skill_ids.reshape(NW, PER_W_IDX).astype(jnp.int32)
    tpad = jnp.pad(embeddings, ((0, 0), (0, PDIM - DIM)))
    out = _gather(idx, tpad)
    return out[:, :DIM].reshape(BATCH, HIST, DIM)
